# Initial kernel scaffold; baseline (speedup 1.0000x reference)
import jax
import jax.numpy as jnp
from jax.experimental import pallas as pl


def kernel(x, guide, w_gl, b_gl, w_ec, g_ec, be_ec, w_pj, g_pj, be_pj, bias):
    raise NotImplementedError("write your pallas kernel here")


# speedup vs baseline: 2.5878x; 2.5878x over previous
import functools

import jax
import jax.numpy as jnp
import numpy as np
from jax.experimental import pallas as pl
from jax.experimental.pallas import tpu as pltpu

B, C1, H, W = 4, 384, 64, 64
C2, NH, EC, GC = 128, 4, 128, 512
N_GUIDE = 80
HC = C2 // NH
HW = H * W
EPS = 1e-5
INV_SQRT_HC = 1.0 / float(np.sqrt(HC))


def _fused_kernel(x_ref, guide_ref, w_gl_ref, w_ec_ref, w_pj_ref, pack_ref,
                  packt_ref, out_ref):
    xf = x_ref[0]
    g = jnp.dot(guide_ref[0], w_gl_ref[...],
                preferred_element_type=jnp.float32) + pack_ref[4:5, :]
    emb = jnp.dot(w_ec_ref[...], xf, preferred_element_type=jnp.float32)
    emb = emb * packt_ref[:, 0:1] + packt_ref[:, 1:2]

    aw_rows = []
    for m in range(NH):
        gm = g[:, m * HC:(m + 1) * HC]
        em = emb[m * HC:(m + 1) * HC, :]
        sm = jnp.dot(gm, em, preferred_element_type=jnp.float32)
        awm = jnp.max(sm, axis=0, keepdims=True)
        awm = awm * INV_SQRT_HC + pack_ref[5:6, m:m + 1]
        aw_rows.append(jax.nn.sigmoid(awm))
    aw = jnp.concatenate(aw_rows, axis=0)

    lane = jax.lax.broadcasted_iota(jnp.int32, (1, HW), 1)
    hh = lane // W
    ww = lane % W
    acc = jnp.zeros((C2, HW), dtype=jnp.float32)
    for k in range(9):
        dy = k // 3 - 1
        dx = k % 3 - 1
        y = jnp.dot(w_pj_ref[k], xf, preferred_element_type=jnp.float32)
        if dy == 0 and dx == 0:
            acc = acc + y
        else:
            y = jnp.roll(y, shift=-(dy * W + dx), axis=1)
            valid = ((hh + dy >= 0) & (hh + dy < H)
                     & (ww + dx >= 0) & (ww + dx < W))
            acc = acc + jnp.where(valid, y, 0.0)
    xp = acc * packt_ref[:, 2:3] + packt_ref[:, 3:4]
    gated = xp.reshape(NH, HC, HW) * aw[:, None, :]
    out_ref[0] = gated.reshape(C2, HW)


@functools.partial(jax.jit, static_argnames=())
def kernel(x, guide, w_gl, b_gl, w_ec, g_ec, be_ec, w_pj, g_pj, be_pj, bias):
    sq = 1.0 / jnp.sqrt(1.0 + EPS)
    xf = x.reshape(B, C1, HW)
    w_ec2 = w_ec[:, :, 0, 0]
    w_pj9 = jnp.transpose(w_pj, (2, 3, 0, 1)).reshape(9, C2, C1)
    pack = jnp.stack([
        g_ec * sq,
        be_ec,
        g_pj * sq,
        be_pj,
        b_gl,
        jnp.pad(bias, (0, EC - NH)),
        jnp.zeros((EC,), jnp.float32),
        jnp.zeros((EC,), jnp.float32),
    ], axis=0)
    packt = pack.T

    out = pl.pallas_call(
        _fused_kernel,
        grid=(B,),
        in_specs=[
            pl.BlockSpec((1, C1, HW), lambda b: (b, 0, 0)),
            pl.BlockSpec((1, N_GUIDE, GC), lambda b: (b, 0, 0)),
            pl.BlockSpec((GC, EC), lambda b: (0, 0)),
            pl.BlockSpec((C2, C1), lambda b: (0, 0)),
            pl.BlockSpec((9, C2, C1), lambda b: (0, 0, 0)),
            pl.BlockSpec((8, EC), lambda b: (0, 0)),
            pl.BlockSpec((EC, 8), lambda b: (0, 0)),
        ],
        out_specs=pl.BlockSpec((1, C2, HW), lambda b: (b, 0, 0)),
        out_shape=jax.ShapeDtypeStruct((B, C2, HW), jnp.float32),
        compiler_params=pltpu.CompilerParams(
            dimension_semantics=("parallel",),
        ),
    )(xf, guide, w_gl, w_ec2, w_pj9, pack, packt)
    return out.reshape(B, C2, H, W)


# speedup vs baseline: 2.5971x; 1.0036x over previous
import functools

import jax
import jax.numpy as jnp
import numpy as np
from jax.experimental import pallas as pl
from jax.experimental.pallas import tpu as pltpu

B, C1, H, W = 4, 384, 64, 64
C2, NH, EC, GC = 128, 4, 128, 512
N_GUIDE = 80
HC = C2 // NH
HW = H * W
EPS = 1e-5
INV_SQRT_HC = 1.0 / float(np.sqrt(HC))


def _fused_kernel(x_ref, guide_ref, w_gl_ref, w_ec_ref, w_pj_ref, pack_ref,
                  packt_ref, out_ref):
    xf = x_ref[0].astype(jnp.bfloat16)
    g = jnp.dot(guide_ref[0], w_gl_ref[...],
                preferred_element_type=jnp.float32) + pack_ref[4:5, :]
    emb = jnp.dot(w_ec_ref[...], xf, preferred_element_type=jnp.float32)
    emb = emb * packt_ref[:, 0:1] + packt_ref[:, 1:2]

    aw_rows = []
    for m in range(NH):
        gm = g[:, m * HC:(m + 1) * HC]
        em = emb[m * HC:(m + 1) * HC, :]
        sm = jnp.dot(gm, em, preferred_element_type=jnp.float32)
        awm = jnp.max(sm, axis=0, keepdims=True)
        awm = awm * INV_SQRT_HC + pack_ref[5:6, m:m + 1]
        aw_rows.append(jax.nn.sigmoid(awm))
    aw = jnp.concatenate(aw_rows, axis=0)

    lane = jax.lax.broadcasted_iota(jnp.int32, (1, HW), 1)
    hh = lane // W
    ww = lane % W
    acc = jnp.zeros((C2, HW), dtype=jnp.float32)
    for k in range(9):
        dy = k // 3 - 1
        dx = k % 3 - 1
        y = jnp.dot(w_pj_ref[k], xf, preferred_element_type=jnp.float32)
        if dy == 0 and dx == 0:
            acc = acc + y
        else:
            y = jnp.roll(y, shift=-(dy * W + dx), axis=1)
            valid = ((hh + dy >= 0) & (hh + dy < H)
                     & (ww + dx >= 0) & (ww + dx < W))
            acc = acc + jnp.where(valid, y, 0.0)
    xp = acc * packt_ref[:, 2:3] + packt_ref[:, 3:4]
    gated = xp.reshape(NH, HC, HW) * aw[:, None, :]
    out_ref[0] = gated.reshape(C2, HW)


@functools.partial(jax.jit, static_argnames=())
def kernel(x, guide, w_gl, b_gl, w_ec, g_ec, be_ec, w_pj, g_pj, be_pj, bias):
    sq = 1.0 / jnp.sqrt(1.0 + EPS)
    xf = x.reshape(B, C1, HW)
    w_ec2 = w_ec[:, :, 0, 0].astype(jnp.bfloat16)
    w_pj9 = jnp.transpose(w_pj, (2, 3, 0, 1)).reshape(9, C2, C1)
    w_pj9 = w_pj9.astype(jnp.bfloat16)
    pack = jnp.stack([
        g_ec * sq,
        be_ec,
        g_pj * sq,
        be_pj,
        b_gl,
        jnp.pad(bias, (0, EC - NH)),
        jnp.zeros((EC,), jnp.float32),
        jnp.zeros((EC,), jnp.float32),
    ], axis=0)
    packt = pack.T

    out = pl.pallas_call(
        _fused_kernel,
        grid=(B,),
        in_specs=[
            pl.BlockSpec((1, C1, HW), lambda b: (b, 0, 0)),
            pl.BlockSpec((1, N_GUIDE, GC), lambda b: (b, 0, 0)),
            pl.BlockSpec((GC, EC), lambda b: (0, 0)),
            pl.BlockSpec((C2, C1), lambda b: (0, 0)),
            pl.BlockSpec((9, C2, C1), lambda b: (0, 0, 0)),
            pl.BlockSpec((8, EC), lambda b: (0, 0)),
            pl.BlockSpec((EC, 8), lambda b: (0, 0)),
        ],
        out_specs=pl.BlockSpec((1, C2, HW), lambda b: (b, 0, 0)),
        out_shape=jax.ShapeDtypeStruct((B, C2, HW), jnp.float32),
        compiler_params=pltpu.CompilerParams(
            dimension_semantics=("parallel",),
        ),
    )(xf, guide, w_gl, w_ec2, w_pj9, pack, packt)
    return out.reshape(B, C2, H, W)


# speedup vs baseline: 2.6494x; 1.0201x over previous
import functools

import jax
import jax.numpy as jnp
import numpy as np
from jax.experimental import pallas as pl
from jax.experimental.pallas import tpu as pltpu

B, C1, H, W = 4, 384, 64, 64
C2, NH, EC, GC = 128, 4, 128, 512
N_GUIDE = 80
HC = C2 // NH
HW = H * W
EPS = 1e-5
INV_SQRT_HC = 1.0 / float(np.sqrt(HC))


def _fused_kernel(x_ref, guide_ref, w_gl_ref, w_big_ref, pack_ref,
                  packt_ref, out_ref):
    xf = x_ref[0].astype(jnp.bfloat16)
    g = jnp.dot(guide_ref[0], w_gl_ref[...],
                preferred_element_type=jnp.float32) + pack_ref[4:5, :]
    big = jnp.dot(w_big_ref[...], xf, preferred_element_type=jnp.float32)
    emb = big[0:C2] * packt_ref[:, 0:1] + packt_ref[:, 1:2]

    head = jax.lax.broadcasted_iota(jnp.int32, (1, EC), 1) // HC
    gbd = jnp.concatenate(
        [jnp.where(head == m, g, 0.0) for m in range(NH)], axis=0)
    s = jnp.dot(gbd.astype(jnp.bfloat16), emb.astype(jnp.bfloat16),
                preferred_element_type=jnp.float32)
    aw_rows = []
    for m in range(NH):
        awm = jnp.max(s[m * N_GUIDE:(m + 1) * N_GUIDE], axis=0,
                      keepdims=True)
        awm = awm * INV_SQRT_HC + pack_ref[5:6, m:m + 1]
        aw_rows.append(jax.nn.sigmoid(awm))
    aw = jnp.concatenate(aw_rows, axis=0)

    lane = jax.lax.broadcasted_iota(jnp.int32, (1, HW), 1)
    hh = lane // W
    ww = lane % W
    acc = jnp.zeros((C2, HW), dtype=jnp.float32)
    for k in range(9):
        dy = k // 3 - 1
        dx = k % 3 - 1
        y = big[C2 + k * C2:C2 + (k + 1) * C2]
        if dy == 0 and dx == 0:
            acc = acc + y
        else:
            y = jnp.roll(y, shift=-(dy * W + dx), axis=1)
            valid = ((hh + dy >= 0) & (hh + dy < H)
                     & (ww + dx >= 0) & (ww + dx < W))
            acc = acc + jnp.where(valid, y, 0.0)
    xp = acc * packt_ref[:, 2:3] + packt_ref[:, 3:4]
    gated = xp.reshape(NH, HC, HW) * aw[:, None, :]
    out_ref[0] = gated.reshape(C2, HW)


@functools.partial(jax.jit, static_argnames=())
def kernel(x, guide, w_gl, b_gl, w_ec, g_ec, be_ec, w_pj, g_pj, be_pj, bias):
    sq = 1.0 / jnp.sqrt(1.0 + EPS)
    xf = x.reshape(B, C1, HW)
    w_ec2 = w_ec[:, :, 0, 0]
    w_pj9 = jnp.transpose(w_pj, (2, 3, 0, 1)).reshape(9 * C2, C1)
    w_big = jnp.concatenate([w_ec2, w_pj9], axis=0).astype(jnp.bfloat16)
    pack = jnp.stack([
        g_ec * sq,
        be_ec,
        g_pj * sq,
        be_pj,
        b_gl,
        jnp.pad(bias, (0, EC - NH)),
        jnp.zeros((EC,), jnp.float32),
        jnp.zeros((EC,), jnp.float32),
    ], axis=0)
    packt = pack.T

    out = pl.pallas_call(
        _fused_kernel,
        grid=(B,),
        in_specs=[
            pl.BlockSpec((1, C1, HW), lambda b: (b, 0, 0)),
            pl.BlockSpec((1, N_GUIDE, GC), lambda b: (b, 0, 0)),
            pl.BlockSpec((GC, EC), lambda b: (0, 0)),
            pl.BlockSpec((10 * C2, C1), lambda b: (0, 0)),
            pl.BlockSpec((8, EC), lambda b: (0, 0)),
            pl.BlockSpec((EC, 8), lambda b: (0, 0)),
        ],
        out_specs=pl.BlockSpec((1, C2, HW), lambda b: (b, 0, 0)),
        out_shape=jax.ShapeDtypeStruct((B, C2, HW), jnp.float32),
        compiler_params=pltpu.CompilerParams(
            dimension_semantics=("parallel",),
        ),
    )(xf, guide, w_gl, w_big, pack, packt)
    return out.reshape(B, C2, H, W)
